# Initial kernel scaffold; baseline (speedup 1.0000x reference)
import jax
import jax.numpy as jnp
from jax.experimental import pallas as pl


def kernel(params, edge_index):
    raise NotImplementedError("write your pallas kernel here")


# speedup vs baseline: 170.1632x; 170.1632x over previous
import functools

import jax
import jax.numpy as jnp
from jax import lax
from jax.experimental import pallas as pl
from jax.experimental.pallas import tpu as pltpu
from jax.experimental.pallas import tpu_sc as plsc

_N_AGENTS = 100000
_N_EDGES = 3200000
_N_TIMESTEPS = 10
_DELTA_T = 1.0
_TAU = 0.1

_ROW = 100
_CHUNKS = _N_EDGES // _ROW
_NWORK = 32
_CPW = _CHUNKS // _NWORK
_K = 8
_WINS = _CPW // _K


def _soft_minimum(a, b, k=2.0):
    b = jnp.broadcast_to(jnp.asarray(b, a.dtype), a.shape)
    return -jax.nn.logsumexp(jnp.stack([-k * a, -k * b], axis=0), axis=0) / k


def _sample_bernoulli_gs(key, probs, tau=_TAU):
    probs = jnp.clip(probs, 1e-10, 1.0 - 1e-10)
    logits = jnp.log(jnp.stack([probs, 1.0 - probs], axis=1))
    g = jax.random.gumbel(key, logits.shape, dtype=logits.dtype)
    y = jax.nn.softmax((logits + g) / tau, axis=1)
    hard = jax.nn.one_hot(jnp.argmax(y, axis=1), 2, dtype=y.dtype)
    out = hard + y - jax.lax.stop_gradient(y)
    return out[:, 0]


@functools.cache
def _build_segsum_sc():
    mesh = plsc.VectorSubcoreMesh(core_axis_name="c", subcore_axis_name="s")

    @functools.partial(
        pl.kernel,
        mesh=mesh,
        out_type=jax.ShapeDtypeStruct((2, _N_AGENTS), jnp.float32),
        scratch_types=[
            pltpu.VMEM_SHARED((_N_AGENTS,), jnp.float32),
            pltpu.VMEM_SHARED((_N_AGENTS,), jnp.float32),
            pltpu.VMEM((_K, _ROW), jnp.int32),
            pltpu.VMEM((_K, _ROW), jnp.int32),
            pltpu.VMEM((_K, _ROW), jnp.float32),
            pltpu.SemaphoreType.DMA,
        ],
    )
    def _segsum_sc(vals_hbm, src_hbm, dst_hbm, zero_hbm, out_hbm,
                   table, acc, srcb, dstb, valsb, gsem):
        c = lax.axis_index("c")
        s = lax.axis_index("s")
        wid = s * 2 + c

        @pl.when(s == 0)
        def _stage():
            pltpu.sync_copy(vals_hbm, table)
            pltpu.sync_copy(zero_hbm, acc)

        plsc.subcore_barrier()

        base0 = wid * _CPW

        def body(w, carry):
            base = base0 + w * _K
            pltpu.sync_copy(src_hbm.at[pl.ds(base, _K)], srcb)
            pltpu.sync_copy(dst_hbm.at[pl.ds(base, _K)], dstb)
            handles = [
                pltpu.async_copy(table.at[srcb.at[j]], valsb.at[j], gsem)
                for j in range(_K)
            ]
            for h in handles:
                h.wait()
            for j in range(_K):
                pltpu.sync_copy(valsb.at[j], acc.at[dstb.at[j]], add=True)
            return carry

        lax.fori_loop(0, _WINS, body, 0)

        plsc.subcore_barrier()

        @pl.when(s == 0)
        def _emit():
            pltpu.sync_copy(acc, out_hbm.at[c])

    return _segsum_sc


def _segment_sum(values, src2d, dst2d, zeros):
    parts = _build_segsum_sc()(values, src2d, dst2d, zeros)
    return parts[0] + parts[1]


def kernel(params, edge_index):
    n = _N_AGENTS
    src2d = edge_index[0].reshape(_CHUNKS, _ROW)
    dst2d = edge_index[1].reshape(_CHUNKS, _ROW)
    zeros = jnp.zeros((n,), jnp.float32)
    aux = jnp.ones(n, dtype=jnp.float32)

    key = jax.random.key(42)
    p0 = _soft_minimum(params, 0.0, 2.0)
    initial_fraction_infected = 10.0 ** p0[2]
    probs = initial_fraction_infected * jnp.ones(n, dtype=jnp.float32)
    key, sk = jax.random.split(key)
    new_infected = _sample_bernoulli_gs(sk, probs)
    infected = new_infected
    susceptible = 1.0 - new_infected
    recovered = jnp.zeros(n, dtype=jnp.float32)
    x = jnp.stack([infected, susceptible, recovered], axis=0)

    n_neighbors = jnp.maximum(1.0, _segment_sum(aux, src2d, dst2d, zeros))

    inf_pd = [x[0].sum() / n]
    sus_pd = [x[1].sum() / n]
    rec_pd = [x[2].sum() / n]
    states_pd = [jnp.argmax(x, axis=0)]
    p = 10.0 ** p0
    gamma = p[0]
    rho = p[1]
    for t in range(_N_TIMESTEPS):
        infected = x[0]
        susceptible = x[1]
        recovered = x[2]
        n_infected_neighbors = (1.0 - infected) * _segment_sum(
            infected, src2d, dst2d, zeros)
        lambda_1 = susceptible
        lambda_2 = rho * recovered
        lambda_ = (lambda_1 + lambda_2) * n_infected_neighbors / n_neighbors * _DELTA_T
        prob_infected_or_relapsed = jnp.clip(1.0 - jnp.exp(-lambda_), 1e-10, 1.0)
        key, sk1, sk2 = jax.random.split(key, 3)
        new_ir = _sample_bernoulli_gs(sk1, prob_infected_or_relapsed)
        prob_recovery = jnp.clip(gamma * infected, 1e-10, 1.0)
        new_rec = _sample_bernoulli_gs(sk2, prob_recovery)
        infected = infected + new_ir - new_rec
        susceptible = susceptible - susceptible * new_ir
        recovered = recovered + new_rec - recovered * new_ir
        x = jnp.stack([infected, susceptible, recovered], axis=0)
        inf_pd.append(x[0].sum() / n)
        sus_pd.append(x[1].sum() / n)
        rec_pd.append(x[2].sum() / n)
        states_pd.append(jnp.argmax(x, axis=0))
    return (jnp.stack(sus_pd), jnp.stack(inf_pd), jnp.stack(rec_pd),
            jnp.stack(states_pd))


# speedup vs baseline: 301.7456x; 1.7733x over previous
import functools

import jax
import jax.numpy as jnp
from jax import lax
from jax.experimental import pallas as pl
from jax.experimental.pallas import tpu as pltpu
from jax.experimental.pallas import tpu_sc as plsc

_N_AGENTS = 100000
_N_EDGES = 3200000
_N_TIMESTEPS = 10
_DELTA_T = 1.0
_TAU = 0.1

_ROW = 100
_CHUNKS = _N_EDGES // _ROW
_NWORK = 32
_CPW = _CHUNKS // _NWORK
_K = 8
_WINS = _CPW // _K
_WEDGE = _K * _ROW
_NBUF = 5


def _soft_minimum(a, b, k=2.0):
    b = jnp.broadcast_to(jnp.asarray(b, a.dtype), a.shape)
    return -jax.nn.logsumexp(jnp.stack([-k * a, -k * b], axis=0), axis=0) / k


def _sample_bernoulli_gs(key, probs, tau=_TAU):
    probs = jnp.clip(probs, 1e-10, 1.0 - 1e-10)
    logits = jnp.log(jnp.stack([probs, 1.0 - probs], axis=1))
    g = jax.random.gumbel(key, logits.shape, dtype=logits.dtype)
    y = jax.nn.softmax((logits + g) / tau, axis=1)
    hard = jax.nn.one_hot(jnp.argmax(y, axis=1), 2, dtype=y.dtype)
    out = hard + y - jax.lax.stop_gradient(y)
    return out[:, 0]


@functools.cache
def _build_segsum_sc():
    mesh = plsc.VectorSubcoreMesh(core_axis_name="c", subcore_axis_name="s")

    @functools.partial(
        pl.kernel,
        mesh=mesh,
        out_type=jax.ShapeDtypeStruct((2, _N_AGENTS), jnp.float32),
        scratch_types=[
            pltpu.VMEM_SHARED((_N_AGENTS,), jnp.float32),
            pltpu.VMEM_SHARED((_N_AGENTS,), jnp.float32),
            *[pltpu.VMEM((_K, _ROW), jnp.int32)
              for _ in range(_NBUF)],
            *[pltpu.VMEM((_K, _ROW), jnp.int32)
              for _ in range(_NBUF)],
            *[pltpu.VMEM((_K, _ROW), jnp.float32)
              for _ in range(_NBUF)],
            pltpu.VMEM((_WEDGE,), jnp.float32),
            pltpu.SemaphoreType.DMA,
            pltpu.SemaphoreType.DMA,
            pltpu.SemaphoreType.DMA,
            pltpu.SemaphoreType.DMA,
            pltpu.SemaphoreType.DMA,
            pltpu.SemaphoreType.DMA,
            pltpu.SemaphoreType.DMA,
            pltpu.SemaphoreType.DMA,
            pltpu.SemaphoreType.DMA,
            pltpu.SemaphoreType.DMA,
            pltpu.SemaphoreType.DMA,
        ],
    )
    def _segsum_sc(vals_hbm, src_hbm, dst_hbm, zero_hbm, out_hbm,
                   table, acc,
                   srcb0, srcb1, srcb2, srcb3, srcb4,
                   dstb0, dstb1, dstb2, dstb3, dstb4,
                   valsb0, valsb1, valsb2, valsb3, valsb4, waitref, gsem,
                   st0, st1, st2, st3, st4, sc0, sc1, sc2, sc3, sc4):
        srcb = [srcb0, srcb1, srcb2, srcb3, srcb4]
        dstb = [dstb0, dstb1, dstb2, dstb3, dstb4]
        valsb = [valsb0, valsb1, valsb2, valsb3, valsb4]
        st = [st0, st1, st2, st3, st4]
        sc = [sc0, sc1, sc2, sc3, sc4]
        c = lax.axis_index("c")
        s = lax.axis_index("s")
        wid = s * 2 + c

        @pl.when(s == 0)
        def _stage_shared():
            pltpu.sync_copy(vals_hbm, table)
            pltpu.sync_copy(zero_hbm, acc)

        plsc.subcore_barrier()

        row0 = wid * _CPW

        def issue_stage(w, b):
            r = row0 + w * _K
            pltpu.async_copy(src_hbm.at[pl.ds(r, _K)], srcb[b], st[b])
            pltpu.async_copy(dst_hbm.at[pl.ds(r, _K)], dstb[b], st[b])

        def wait_stage(b):
            pltpu.make_async_copy(src_hbm.at[pl.ds(0, _K)],
                                  srcb[b], st[b]).wait()
            pltpu.make_async_copy(dst_hbm.at[pl.ds(0, _K)],
                                  dstb[b], st[b]).wait()

        def drain_scatters(b):
            pltpu.make_async_copy(zero_hbm.at[pl.ds(0, _WEDGE)],
                                  waitref, sc[b]).wait()

        def process(w, b):
            ghs = [pltpu.async_copy(table.at[srcb[b].at[j]],
                                    valsb[b].at[j], gsem)
                   for j in range(_K)]
            for gh in ghs:
                gh.wait()
            for j in range(_K):
                pltpu.async_copy(valsb[b].at[j],
                                 acc.at[dstb[b].at[j]], sc[b], add=True)

        for b in range(_NBUF - 1):
            issue_stage(b, b)

        def super_body(m, carry):
            for u in range(_NBUF):
                w = m * _NBUF + u
                b = u
                bprev = (u - 1) % _NBUF
                wait_stage(b)
                if u == 0:
                    @pl.when(m >= 1)
                    def _():
                        drain_scatters(bprev)
                    issue_stage(w + _NBUF - 1, bprev)
                else:
                    drain_scatters(bprev)

                    @pl.when(m < _WINS // _NBUF - 1)
                    def _():
                        issue_stage(w + _NBUF - 1, bprev)
                process(w, b)
            return carry

        lax.fori_loop(0, _WINS // _NBUF, super_body, 0)

        drain_scatters((_WINS - 1) % _NBUF)

        plsc.subcore_barrier()

        @pl.when(s == 0)
        def _emit():
            pltpu.sync_copy(acc, out_hbm.at[c])

    return _segsum_sc


def _segment_sum(values, src2d, dst2d, zeros):
    parts = _build_segsum_sc()(values, src2d, dst2d, zeros)
    return parts[0] + parts[1]


def kernel(params, edge_index):
    n = _N_AGENTS
    src2d = edge_index[0].reshape(_CHUNKS, _ROW)
    dst2d = edge_index[1].reshape(_CHUNKS, _ROW)
    zeros = jnp.zeros((n,), jnp.float32)
    aux = jnp.ones(n, dtype=jnp.float32)

    key = jax.random.key(42)
    p0 = _soft_minimum(params, 0.0, 2.0)
    initial_fraction_infected = 10.0 ** p0[2]
    probs = initial_fraction_infected * jnp.ones(n, dtype=jnp.float32)
    key, sk = jax.random.split(key)
    new_infected = _sample_bernoulli_gs(sk, probs)
    infected = new_infected
    susceptible = 1.0 - new_infected
    recovered = jnp.zeros(n, dtype=jnp.float32)
    x = jnp.stack([infected, susceptible, recovered], axis=0)

    n_neighbors = jnp.maximum(1.0, _segment_sum(aux, src2d, dst2d, zeros))

    inf_pd = [x[0].sum() / n]
    sus_pd = [x[1].sum() / n]
    rec_pd = [x[2].sum() / n]
    states_pd = [jnp.argmax(x, axis=0)]
    p = 10.0 ** p0
    gamma = p[0]
    rho = p[1]
    for t in range(_N_TIMESTEPS):
        infected = x[0]
        susceptible = x[1]
        recovered = x[2]
        n_infected_neighbors = (1.0 - infected) * _segment_sum(
            infected, src2d, dst2d, zeros)
        lambda_1 = susceptible
        lambda_2 = rho * recovered
        lambda_ = (lambda_1 + lambda_2) * n_infected_neighbors / n_neighbors * _DELTA_T
        prob_infected_or_relapsed = jnp.clip(1.0 - jnp.exp(-lambda_), 1e-10, 1.0)
        key, sk1, sk2 = jax.random.split(key, 3)
        new_ir = _sample_bernoulli_gs(sk1, prob_infected_or_relapsed)
        prob_recovery = jnp.clip(gamma * infected, 1e-10, 1.0)
        new_rec = _sample_bernoulli_gs(sk2, prob_recovery)
        infected = infected + new_ir - new_rec
        susceptible = susceptible - susceptible * new_ir
        recovered = recovered + new_rec - recovered * new_ir
        x = jnp.stack([infected, susceptible, recovered], axis=0)
        inf_pd.append(x[0].sum() / n)
        sus_pd.append(x[1].sum() / n)
        rec_pd.append(x[2].sum() / n)
        states_pd.append(jnp.argmax(x, axis=0))
    return (jnp.stack(sus_pd), jnp.stack(inf_pd), jnp.stack(rec_pd),
            jnp.stack(states_pd))


# speedup vs baseline: 319.0406x; 1.0573x over previous
import functools

import jax
import jax.numpy as jnp
from jax import lax
from jax.experimental import pallas as pl
from jax.experimental.pallas import tpu as pltpu
from jax.experimental.pallas import tpu_sc as plsc

_N_AGENTS = 100000
_N_EDGES = 3200000
_N_TIMESTEPS = 10
_DELTA_T = 1.0
_TAU = 0.1

_ROW = 100
_CHUNKS = _N_EDGES // _ROW
_NWORK = 32
_CPW = _CHUNKS // _NWORK
_K = 8
_WINS = _CPW // _K
_WEDGE = _K * _ROW
_NBUF = 5


def _soft_minimum(a, b, k=2.0):
    b = jnp.broadcast_to(jnp.asarray(b, a.dtype), a.shape)
    return -jax.nn.logsumexp(jnp.stack([-k * a, -k * b], axis=0), axis=0) / k


def _sample_bernoulli_gs(g, probs, tau=_TAU):
    probs = jnp.clip(probs, 1e-10, 1.0 - 1e-10)
    logits = jnp.log(jnp.stack([probs, 1.0 - probs], axis=1))
    y = jax.nn.softmax((logits + g) / tau, axis=1)
    hard = jax.nn.one_hot(jnp.argmax(y, axis=1), 2, dtype=y.dtype)
    out = hard + y - jax.lax.stop_gradient(y)
    return out[:, 0]


def _make_segsum_body(with_degree):
    def body(vals_hbm, src_hbm, dst_hbm, zero_hbm, ones_hbm, out_hbm,
             table, acc, acc2, onesb,
             srcb0, srcb1, srcb2, srcb3, srcb4,
             dstb0, dstb1, dstb2, dstb3, dstb4,
             valsb0, valsb1, valsb2, valsb3, valsb4, waitref, gsem,
             st0, st1, st2, st3, st4, sc0, sc1, sc2, sc3, sc4):
        srcb = [srcb0, srcb1, srcb2, srcb3, srcb4]
        dstb = [dstb0, dstb1, dstb2, dstb3, dstb4]
        valsb = [valsb0, valsb1, valsb2, valsb3, valsb4]
        st = [st0, st1, st2, st3, st4]
        sc = [sc0, sc1, sc2, sc3, sc4]
        c = lax.axis_index("c")
        s = lax.axis_index("s")
        wid = s * 2 + c
        row0 = wid * _CPW

        def issue_stage(w, b):
            r = row0 + w * _K
            pltpu.async_copy(src_hbm.at[pl.ds(r, _K)], srcb[b], st[b])
            pltpu.async_copy(dst_hbm.at[pl.ds(r, _K)], dstb[b], st[b])

        def wait_stage(b):
            pltpu.make_async_copy(src_hbm.at[pl.ds(0, _K)],
                                  srcb[b], st[b]).wait()
            pltpu.make_async_copy(dst_hbm.at[pl.ds(0, _K)],
                                  dstb[b], st[b]).wait()

        def drain_scatters(b):
            pltpu.make_async_copy(zero_hbm.at[pl.ds(0, _WEDGE)],
                                  waitref, sc[b]).wait()
            if with_degree:
                pltpu.make_async_copy(zero_hbm.at[pl.ds(0, _WEDGE)],
                                      waitref, sc[b]).wait()

        def process(w, b):
            if with_degree:
                for j in range(_K):
                    pltpu.async_copy(onesb.at[j],
                                     acc2.at[dstb[b].at[j]], sc[b], add=True)
            ghs = [pltpu.async_copy(table.at[srcb[b].at[j]],
                                    valsb[b].at[j], gsem)
                   for j in range(_K)]
            for gh in ghs:
                gh.wait()
            for j in range(_K):
                pltpu.async_copy(valsb[b].at[j],
                                 acc.at[dstb[b].at[j]], sc[b], add=True)

        for b in range(_NBUF - 1):
            issue_stage(b, b)

        @pl.when(s == 0)
        def _stage_shared():
            pltpu.sync_copy(vals_hbm, table)
            pltpu.sync_copy(zero_hbm, acc)
            if with_degree:
                pltpu.sync_copy(zero_hbm, acc2)

        if with_degree:
            pltpu.sync_copy(ones_hbm, onesb)

        plsc.subcore_barrier()

        def super_body(m, carry):
            for u in range(_NBUF):
                w = m * _NBUF + u
                b = u
                bprev = (u - 1) % _NBUF
                wait_stage(b)
                if u == 0:
                    @pl.when(m >= 1)
                    def _():
                        drain_scatters(bprev)
                    issue_stage(w + _NBUF - 1, bprev)
                else:
                    drain_scatters(bprev)

                    @pl.when(m < _WINS // _NBUF - 1)
                    def _():
                        issue_stage(w + _NBUF - 1, bprev)
                process(w, b)
            return carry

        lax.fori_loop(0, _WINS // _NBUF, super_body, 0)

        drain_scatters((_WINS - 1) % _NBUF)

        plsc.subcore_barrier()

        @pl.when(s == 0)
        def _emit():
            pltpu.sync_copy(acc, out_hbm.at[c])
            if with_degree:
                pltpu.sync_copy(acc2, out_hbm.at[2 + c])

    return body


@functools.cache
def _build_segsum_sc(with_degree):
    mesh = plsc.VectorSubcoreMesh(core_axis_name="c", subcore_axis_name="s")
    n_out = 4 if with_degree else 2
    return pl.kernel(
        _make_segsum_body(with_degree),
        mesh=mesh,
        out_type=jax.ShapeDtypeStruct((n_out, _N_AGENTS), jnp.float32),
        scratch_types=[
            pltpu.VMEM_SHARED((_N_AGENTS,), jnp.float32),
            pltpu.VMEM_SHARED((_N_AGENTS,), jnp.float32),
            pltpu.VMEM_SHARED((_N_AGENTS,), jnp.float32),
            pltpu.VMEM((_K, _ROW), jnp.float32),
            *[pltpu.VMEM((_K, _ROW), jnp.int32)
              for _ in range(_NBUF)],
            *[pltpu.VMEM((_K, _ROW), jnp.int32)
              for _ in range(_NBUF)],
            *[pltpu.VMEM((_K, _ROW), jnp.float32)
              for _ in range(_NBUF)],
            pltpu.VMEM((_WEDGE,), jnp.float32),
            pltpu.SemaphoreType.DMA,
            pltpu.SemaphoreType.DMA,
            pltpu.SemaphoreType.DMA,
            pltpu.SemaphoreType.DMA,
            pltpu.SemaphoreType.DMA,
            pltpu.SemaphoreType.DMA,
            pltpu.SemaphoreType.DMA,
            pltpu.SemaphoreType.DMA,
            pltpu.SemaphoreType.DMA,
            pltpu.SemaphoreType.DMA,
            pltpu.SemaphoreType.DMA,
        ],
    )


def kernel(params, edge_index):
    n = _N_AGENTS
    src2d = edge_index[0].reshape(_CHUNKS, _ROW)
    dst2d = edge_index[1].reshape(_CHUNKS, _ROW)
    zeros = jnp.zeros((n,), jnp.float32)
    ones_rows = jnp.ones((_K, _ROW), jnp.float32)

    key = jax.random.key(42)
    key, sk = jax.random.split(key)
    g_init = jax.random.gumbel(sk, (n, 2), dtype=jnp.float32)
    g_steps = []
    for _ in range(_N_TIMESTEPS):
        key, sk1, sk2 = jax.random.split(key, 3)
        g_steps.append((jax.random.gumbel(sk1, (n, 2), dtype=jnp.float32),
                        jax.random.gumbel(sk2, (n, 2), dtype=jnp.float32)))

    p0 = _soft_minimum(params, 0.0, 2.0)
    initial_fraction_infected = 10.0 ** p0[2]
    probs = initial_fraction_infected * jnp.ones(n, dtype=jnp.float32)
    new_infected = _sample_bernoulli_gs(g_init, probs)
    infected = new_infected
    susceptible = 1.0 - new_infected
    recovered = jnp.zeros(n, dtype=jnp.float32)
    x = jnp.stack([infected, susceptible, recovered], axis=0)

    p = 10.0 ** p0
    gamma = p[0]
    rho = p[1]
    xs = [x]
    n_neighbors = None
    for t in range(_N_TIMESTEPS):
        infected = x[0]
        susceptible = x[1]
        recovered = x[2]
        if t == 0:
            parts = _build_segsum_sc(True)(
                infected, src2d, dst2d, zeros, ones_rows)
            seg = parts[0] + parts[1]
            n_neighbors = jnp.maximum(1.0, parts[2] + parts[3])
        else:
            parts = _build_segsum_sc(False)(
                infected, src2d, dst2d, zeros, ones_rows)
            seg = parts[0] + parts[1]
        n_infected_neighbors = (1.0 - infected) * seg
        lambda_1 = susceptible
        lambda_2 = rho * recovered
        lambda_ = (lambda_1 + lambda_2) * n_infected_neighbors / n_neighbors * _DELTA_T
        prob_infected_or_relapsed = jnp.clip(1.0 - jnp.exp(-lambda_), 1e-10, 1.0)
        g1, g2 = g_steps[t]
        new_ir = _sample_bernoulli_gs(g1, prob_infected_or_relapsed)
        prob_recovery = jnp.clip(gamma * infected, 1e-10, 1.0)
        new_rec = _sample_bernoulli_gs(g2, prob_recovery)
        infected = infected + new_ir - new_rec
        susceptible = susceptible - susceptible * new_ir
        recovered = recovered + new_rec - recovered * new_ir
        x = jnp.stack([infected, susceptible, recovered], axis=0)
        xs.append(x)

    X = jnp.stack(xs)
    sums = X.sum(axis=2) / n
    states = jnp.argmax(X, axis=1)
    return (sums[:, 1], sums[:, 0], sums[:, 2], states)


# speedup vs baseline: 322.3833x; 1.0105x over previous
import functools

import jax
import jax.numpy as jnp
from jax import lax
from jax.experimental import pallas as pl
from jax.experimental.pallas import tpu as pltpu
from jax.experimental.pallas import tpu_sc as plsc

_N_AGENTS = 100000
_N_EDGES = 3200000
_N_TIMESTEPS = 10
_DELTA_T = 1.0
_TAU = 0.1

_ROW = 100
_CHUNKS = _N_EDGES // _ROW
_NWORK = 32
_CPW = _CHUNKS // _NWORK
_K = 40
_WINS = _CPW // _K
_WEDGE = _K * _ROW
_NBUF = 5


def _soft_minimum(a, b, k=2.0):
    b = jnp.broadcast_to(jnp.asarray(b, a.dtype), a.shape)
    return -jax.nn.logsumexp(jnp.stack([-k * a, -k * b], axis=0), axis=0) / k


def _sample_bernoulli_gs(g, probs, tau=_TAU):
    probs = jnp.clip(probs, 1e-10, 1.0 - 1e-10)
    logits = jnp.log(jnp.stack([probs, 1.0 - probs], axis=1))
    y = jax.nn.softmax((logits + g) / tau, axis=1)
    hard = jax.nn.one_hot(jnp.argmax(y, axis=1), 2, dtype=y.dtype)
    out = hard + y - jax.lax.stop_gradient(y)
    return out[:, 0]


def _make_segsum_body(with_degree):
    def body(vals_hbm, src_hbm, dst_hbm, zero_hbm, ones_hbm, out_hbm,
             table, acc, acc2, onesb,
             srcb0, srcb1, srcb2, srcb3, srcb4,
             dstb0, dstb1, dstb2, dstb3, dstb4,
             valsb0, valsb1, valsb2, valsb3, valsb4, waitref, gsem,
             st0, st1, st2, st3, st4, sc0, sc1, sc2, sc3, sc4):
        srcb = [srcb0, srcb1, srcb2, srcb3, srcb4]
        dstb = [dstb0, dstb1, dstb2, dstb3, dstb4]
        valsb = [valsb0, valsb1, valsb2, valsb3, valsb4]
        st = [st0, st1, st2, st3, st4]
        sc = [sc0, sc1, sc2, sc3, sc4]
        c = lax.axis_index("c")
        s = lax.axis_index("s")
        wid = s * 2 + c
        row0 = wid * _CPW

        def issue_stage(w, b):
            r = row0 + w * _K
            pltpu.async_copy(src_hbm.at[pl.ds(r, _K)], srcb[b], st[b])
            pltpu.async_copy(dst_hbm.at[pl.ds(r, _K)], dstb[b], st[b])

        def wait_stage(b):
            pltpu.make_async_copy(src_hbm.at[pl.ds(0, _K)],
                                  srcb[b], st[b]).wait()
            pltpu.make_async_copy(dst_hbm.at[pl.ds(0, _K)],
                                  dstb[b], st[b]).wait()

        def drain_scatters(b):
            pltpu.make_async_copy(zero_hbm.at[pl.ds(0, _WEDGE)],
                                  waitref, sc[b]).wait()
            if with_degree:
                pltpu.make_async_copy(zero_hbm.at[pl.ds(0, _WEDGE)],
                                      waitref, sc[b]).wait()

        def process(w, b):
            if with_degree:
                for j in range(_K):
                    pltpu.async_copy(onesb.at[j],
                                     acc2.at[dstb[b].at[j]], sc[b], add=True)
            ghs = [pltpu.async_copy(table.at[srcb[b].at[j]],
                                    valsb[b].at[j], gsem)
                   for j in range(_K)]
            for j in range(_K):
                ghs[j].wait()
                pltpu.async_copy(valsb[b].at[j],
                                 acc.at[dstb[b].at[j]], sc[b], add=True)

        for b in range(_NBUF - 1):
            issue_stage(b, b)

        @pl.when(s == 0)
        def _stage_shared():
            pltpu.sync_copy(vals_hbm, table)
            pltpu.sync_copy(zero_hbm, acc)
            if with_degree:
                pltpu.sync_copy(zero_hbm, acc2)

        if with_degree:
            pltpu.sync_copy(ones_hbm, onesb)

        plsc.subcore_barrier()

        def super_body(m, carry):
            for u in range(_NBUF):
                w = m * _NBUF + u
                b = u
                bprev = (u - 1) % _NBUF
                wait_stage(b)
                if u == 0:
                    @pl.when(m >= 1)
                    def _():
                        drain_scatters(bprev)
                    issue_stage(w + _NBUF - 1, bprev)
                else:
                    drain_scatters(bprev)

                    @pl.when(m < _WINS // _NBUF - 1)
                    def _():
                        issue_stage(w + _NBUF - 1, bprev)
                process(w, b)
            return carry

        lax.fori_loop(0, _WINS // _NBUF, super_body, 0)

        drain_scatters((_WINS - 1) % _NBUF)

        plsc.subcore_barrier()

        @pl.when(s == 0)
        def _emit():
            pltpu.sync_copy(acc, out_hbm.at[c])
            if with_degree:
                pltpu.sync_copy(acc2, out_hbm.at[2 + c])

    return body


@functools.cache
def _build_segsum_sc(with_degree):
    mesh = plsc.VectorSubcoreMesh(core_axis_name="c", subcore_axis_name="s")
    n_out = 4 if with_degree else 2
    return pl.kernel(
        _make_segsum_body(with_degree),
        mesh=mesh,
        out_type=jax.ShapeDtypeStruct((n_out, _N_AGENTS), jnp.float32),
        scratch_types=[
            pltpu.VMEM_SHARED((_N_AGENTS,), jnp.float32),
            pltpu.VMEM_SHARED((_N_AGENTS,), jnp.float32),
            pltpu.VMEM_SHARED((_N_AGENTS,), jnp.float32),
            pltpu.VMEM((_K, _ROW), jnp.float32),
            *[pltpu.VMEM((_K, _ROW), jnp.int32)
              for _ in range(_NBUF)],
            *[pltpu.VMEM((_K, _ROW), jnp.int32)
              for _ in range(_NBUF)],
            *[pltpu.VMEM((_K, _ROW), jnp.float32)
              for _ in range(_NBUF)],
            pltpu.VMEM((_WEDGE,), jnp.float32),
            pltpu.SemaphoreType.DMA,
            pltpu.SemaphoreType.DMA,
            pltpu.SemaphoreType.DMA,
            pltpu.SemaphoreType.DMA,
            pltpu.SemaphoreType.DMA,
            pltpu.SemaphoreType.DMA,
            pltpu.SemaphoreType.DMA,
            pltpu.SemaphoreType.DMA,
            pltpu.SemaphoreType.DMA,
            pltpu.SemaphoreType.DMA,
            pltpu.SemaphoreType.DMA,
        ],
    )


def kernel(params, edge_index):
    n = _N_AGENTS
    src2d = edge_index[0].reshape(_CHUNKS, _ROW)
    dst2d = edge_index[1].reshape(_CHUNKS, _ROW)
    zeros = jnp.zeros((n,), jnp.float32)
    ones_rows = jnp.ones((_K, _ROW), jnp.float32)

    key = jax.random.key(42)
    key, sk = jax.random.split(key)
    g_init = jax.random.gumbel(sk, (n, 2), dtype=jnp.float32)
    g_steps = []
    for _ in range(_N_TIMESTEPS):
        key, sk1, sk2 = jax.random.split(key, 3)
        g_steps.append((jax.random.gumbel(sk1, (n, 2), dtype=jnp.float32),
                        jax.random.gumbel(sk2, (n, 2), dtype=jnp.float32)))

    p0 = _soft_minimum(params, 0.0, 2.0)
    initial_fraction_infected = 10.0 ** p0[2]
    probs = initial_fraction_infected * jnp.ones(n, dtype=jnp.float32)
    new_infected = _sample_bernoulli_gs(g_init, probs)
    infected = new_infected
    susceptible = 1.0 - new_infected
    recovered = jnp.zeros(n, dtype=jnp.float32)
    x = jnp.stack([infected, susceptible, recovered], axis=0)

    p = 10.0 ** p0
    gamma = p[0]
    rho = p[1]
    xs = [x]
    n_neighbors = None
    for t in range(_N_TIMESTEPS):
        infected = x[0]
        susceptible = x[1]
        recovered = x[2]
        if t == 0:
            parts = _build_segsum_sc(True)(
                infected, src2d, dst2d, zeros, ones_rows)
            seg = parts[0] + parts[1]
            n_neighbors = jnp.maximum(1.0, parts[2] + parts[3])
        else:
            parts = _build_segsum_sc(False)(
                infected, src2d, dst2d, zeros, ones_rows)
            seg = parts[0] + parts[1]
        n_infected_neighbors = (1.0 - infected) * seg
        lambda_1 = susceptible
        lambda_2 = rho * recovered
        lambda_ = (lambda_1 + lambda_2) * n_infected_neighbors / n_neighbors * _DELTA_T
        prob_infected_or_relapsed = jnp.clip(1.0 - jnp.exp(-lambda_), 1e-10, 1.0)
        g1, g2 = g_steps[t]
        new_ir = _sample_bernoulli_gs(g1, prob_infected_or_relapsed)
        prob_recovery = jnp.clip(gamma * infected, 1e-10, 1.0)
        new_rec = _sample_bernoulli_gs(g2, prob_recovery)
        infected = infected + new_ir - new_rec
        susceptible = susceptible - susceptible * new_ir
        recovered = recovered + new_rec - recovered * new_ir
        x = jnp.stack([infected, susceptible, recovered], axis=0)
        xs.append(x)

    X = jnp.stack(xs)
    sums = X.sum(axis=2) / n
    states = jnp.argmax(X, axis=1)
    return (sums[:, 1], sums[:, 0], sums[:, 2], states)
